# baseline (speedup 1.0000x reference)
import jax
import jax.numpy as jnp
from jax import lax
from jax.experimental import pallas as pl
from jax.experimental.pallas import tpu as pltpu
from jax.experimental.pallas import tpu_sc as plsc

N = 100000
H = 128
S = 1024
L = 16
NC = 2
NS = 16
NW = NC * NS
SEG_PER = S // NW
R = H // L

CA = 6272
CA_LAST = N - (NS - 1) * CA
STEPS = CA // L
STEPS_LAST = CA_LAST // L
OFF_LEN = 1032

K = 128

_mesh = plsc.VectorSubcoreMesh(core_axis_name="c", subcore_axis_name="s")
_cparams = pltpu.CompilerParams(needs_layout_passes=False,
                                use_tc_tiling_on_sc=False)


def _mix_body(x_hbm, batch_hbm, w_hbm, out_hbm,
              chunk_v, eloc_v, shared_hist, shared_off, shared_cmax,
              stripe_v, cmax_v, cm_v, offa_v,
              offw_v, w_v, buf3_v, buft_v, obuf_v, sem0, sem1, semt):
    cid = lax.axis_index("c")
    sid = lax.axis_index("s")
    wid = sid * NC + cid
    base = wid * SEG_PER
    i16 = lax.iota(jnp.int32, L)

    def zstep(k, _):
        eloc_v[pl.ds(k * L, L)] = jnp.zeros((L,), jnp.int32)
        return 0
    lax.fori_loop(0, S // L, zstep, 0)

    last = sid == NS - 1

    @pl.when(jnp.logical_not(last))
    def _():
        pltpu.sync_copy(batch_hbm.at[pl.ds(sid * CA, CA + L)],
                        chunk_v.at[pl.ds(0, CA + L)])

    @pl.when(last)
    def _():
        pltpu.sync_copy(batch_hbm.at[pl.ds((NS - 1) * CA, CA_LAST)],
                        chunk_v.at[pl.ds(0, CA_LAST)])

    gbase = sid * CA
    nsteps = jnp.where(last, STEPS_LAST, STEPS)
    nquads = nsteps // 4

    def stepn(n):
        def body(q, _):
            i = n * q
            val = gbase + i * L + 1 + i16
            for u in range(n):
                a = chunk_v[pl.ds((i + u) * L, L)]
                b = chunk_v[pl.ds((i + u) * L + 1, L)]
                plsc.store_scatter(eloc_v, [a], val + u * L, mask=a != b)
            return 0
        return body
    lax.fori_loop(0, nquads, stepn(4), 0)
    lax.fori_loop(4 * nquads, nsteps, stepn(1), 0)

    @pl.when(last)
    def _():
        vlast = chunk_v[pl.ds(CA_LAST - L, L)]
        blast = jnp.max(jnp.where(i16 == L - 1, vlast, 0))
        plsc.store_scatter(eloc_v, [blast + jnp.zeros((L,), jnp.int32)],
                           jnp.full((L,), N, jnp.int32), mask=i16 == 0)

    pltpu.sync_copy(eloc_v, shared_hist.at[sid])
    plsc.subcore_barrier()

    pltpu.sync_copy(shared_hist.at[:, pl.ds(sid * 64, 64)], stripe_v)
    for k in range(4):
        m = stripe_v[0, pl.ds(k * L, L)]
        for rr in range(1, NS):
            m = jnp.maximum(m, stripe_v[rr, pl.ds(k * L, L)])
        cmax_v[pl.ds(k * L, L)] = m
    pltpu.sync_copy(cmax_v, shared_cmax.at[pl.ds(sid * 64, 64)])
    plsc.subcore_barrier()

    @pl.when(sid == 0)
    def _():
        pltpu.sync_copy(shared_cmax, cm_v)

        def red(k, carry):
            c = jnp.maximum(plsc.cummax(cm_v[pl.ds(k * L, L)]), carry)
            plsc.store_scatter(offa_v, [k * L + 1 + i16], c)
            return jnp.max(c)
        lax.fori_loop(0, S // L, red, jnp.int32(0))

        plsc.store_scatter(offa_v, [i16], jnp.zeros((L,), jnp.int32),
                           mask=i16 == 0)
        plsc.store_scatter(offa_v, [jnp.int32(S + 1) + i16],
                           jnp.full((L,), N, jnp.int32),
                           mask=i16 < OFF_LEN - (S + 1))
        pltpu.sync_copy(offa_v, shared_off)

    plsc.subcore_barrier()

    pltpu.sync_copy(shared_off.at[pl.ds(base, 40)], offw_v)
    pltpu.sync_copy(w_hbm, w_v)
    wv = plsc.load_gather(w_v, [jnp.minimum(i16, 2)])
    w0 = jnp.sum(jnp.where(i16 == 0, wv, 0.0))
    w1 = jnp.sum(jnp.where(i16 == 1, wv, 0.0))
    w2 = jnp.sum(jnp.where(i16 == 2, wv, 0.0))

    def get_off(j):
        chunk = offw_v[pl.ds((j // L) * L, L)]
        return jnp.max(jnp.where(i16 == j % L, chunk, 0))

    acc0 = (tuple(jnp.zeros((L,), jnp.float32) for _ in range(R)),
            tuple(jnp.full((L,), -3.0e38, jnp.float32) for _ in range(R)))

    def row_loop(ref, a, b, bufbase, accs):
        def rbody1(row, ac):
            s_t, m_t = ac
            bi = row - bufbase
            s_l, m_l = [], []
            for r in range(R):
                v = ref[bi, pl.ds(r * L, L)]
                s_l.append(s_t[r] + v)
                m_l.append(jnp.maximum(m_t[r], v))
            return (tuple(s_l), tuple(m_l))

        def rbody8(k, ac):
            s_t, m_t = ac
            bi = (a - bufbase) + 8 * k
            s_l, m_l = [], []
            for r in range(R):
                v = [ref[bi + u, pl.ds(r * L, L)] for u in range(8)]
                s_l.append(s_t[r] + (((v[0] + v[1]) + (v[2] + v[3]))
                                     + ((v[4] + v[5]) + (v[6] + v[7]))))
                m01 = jnp.maximum(v[0], v[1])
                m23 = jnp.maximum(v[2], v[3])
                m45 = jnp.maximum(v[4], v[5])
                m67 = jnp.maximum(v[6], v[7])
                m_l.append(jnp.maximum(m_t[r], jnp.maximum(
                    jnp.maximum(m01, m23), jnp.maximum(m45, m67))))
            return (tuple(s_l), tuple(m_l))

        n8 = (b - a) // 8
        accs = lax.fori_loop(0, n8, rbody8, accs)
        return lax.fori_loop(a + 8 * n8, b, rbody1, accs)

    def consume(ref, chi, bufbase, st):
        def w_cond(t):
            s, _, send, _, _ = t
            return jnp.logical_and(s < SEG_PER, send <= chi)

        def w_body(t):
            s, sst, send, cur, accs = t
            accs = row_loop(ref, cur, send, bufbase, accs)
            s_t, m_t = accs
            cnt = send - sst
            cntf = cnt.astype(jnp.float32)
            posf = jnp.where(cnt > 0, jnp.float32(1.0), jnp.float32(0.0))
            denom = jnp.maximum(cntf, 1.0)
            for r in range(R):
                obuf_v[s, pl.ds(r * L, L)] = (w0 * (s_t[r] / denom)
                                              + w1 * (m_t[r] * posf)
                                              + w2 * s_t[r])
            s2 = s + 1
            return (s2, send, get_off(s2 + 1), send, acc0)

        s, sst, send, cur, accs = lax.while_loop(w_cond, w_body, st)
        accs = row_loop(ref, cur, chi, bufbase, accs)
        return (s, sst, send, chi, accs)

    rs = get_off(jnp.int32(0))
    re = get_off(jnp.int32(SEG_PER))
    nf = (re - rs) // K
    ot = jnp.maximum(re - K, 0)

    @pl.when(nf > 0)
    def _():
        pltpu.async_copy(x_hbm.at[pl.ds(rs, K)], buf3_v.at[pl.ds(0, K)], sem0)

    @pl.when(nf > 1)
    def _():
        pltpu.async_copy(x_hbm.at[pl.ds(rs + K, K)],
                         buf3_v.at[pl.ds(K, K)], sem1)

    pltpu.async_copy(x_hbm.at[pl.ds(ot, K)], buft_v, semt)

    st0 = (jnp.int32(0), rs, get_off(jnp.int32(1)), rs, acc0)

    def chunk_body(c, st):
        par = lax.rem(c, 2)

        @pl.when(par == 0)
        def _():
            pltpu.make_async_copy(x_hbm.at[pl.ds(0, K)],
                                  buf3_v.at[pl.ds(0, K)], sem0).wait()

        @pl.when(par == 1)
        def _():
            pltpu.make_async_copy(x_hbm.at[pl.ds(0, K)],
                                  buf3_v.at[pl.ds(K, K)], sem1).wait()

        clo = rs + c * K
        st = consume(buf3_v, clo + K, clo - par * K, st)

        nxt = clo + 2 * K
        issue = nxt < rs + nf * K

        @pl.when(jnp.logical_and(issue, par == 0))
        def _():
            pltpu.async_copy(x_hbm.at[pl.ds(nxt, K)],
                             buf3_v.at[pl.ds(0, K)], sem0)

        @pl.when(jnp.logical_and(issue, par == 1))
        def _():
            pltpu.async_copy(x_hbm.at[pl.ds(nxt, K)],
                             buf3_v.at[pl.ds(K, K)], sem1)

        return st

    st = lax.fori_loop(0, nf, chunk_body, st0)

    pltpu.make_async_copy(x_hbm.at[pl.ds(0, K)], buft_v, semt).wait()
    consume(buft_v, re, ot, st)

    pltpu.sync_copy(obuf_v, out_hbm.at[pl.ds(base, SEG_PER)])


_mix_call = pl.kernel(
    _mix_body,
    out_type=jax.ShapeDtypeStruct((S, H), jnp.float32),
    mesh=_mesh,
    compiler_params=_cparams,
    scratch_types=[
        pltpu.VMEM((CA + L,), jnp.int32),
        pltpu.VMEM((S,), jnp.int32),
        pltpu.VMEM_SHARED((NS, S), jnp.int32),
        pltpu.VMEM_SHARED((OFF_LEN,), jnp.int32),
        pltpu.VMEM_SHARED((S,), jnp.int32),
        pltpu.VMEM((NS, 64), jnp.int32),
        pltpu.VMEM((64,), jnp.int32),
        pltpu.VMEM((S,), jnp.int32),
        pltpu.VMEM((OFF_LEN,), jnp.int32),
        pltpu.VMEM((40,), jnp.int32),
        pltpu.VMEM((3,), jnp.float32),
        pltpu.VMEM((2 * K, H), jnp.float32),
        pltpu.VMEM((K, H), jnp.float32),
        pltpu.VMEM((SEG_PER, H), jnp.float32),
        pltpu.SemaphoreType.DMA,
        pltpu.SemaphoreType.DMA,
        pltpu.SemaphoreType.DMA,
    ],
)


def kernel(x, batch, mask, weights):
    del mask
    return _mix_call(x, batch, weights)


# speedup vs baseline: 1.0330x; 1.0330x over previous
import jax
import jax.numpy as jnp
from jax import lax
from jax.experimental import pallas as pl
from jax.experimental.pallas import tpu as pltpu
from jax.experimental.pallas import tpu_sc as plsc

N = 100000
H = 128
S = 1024
L = 16
NC = 2
NS = 16
NW = NC * NS
SEG_PER = S // NW
R = H // L

CA = 6272
CA_LAST = N - (NS - 1) * CA
STEPS = CA // L
STEPS_LAST = CA_LAST // L
OFF_LEN = 1032

K = 128

_mesh = plsc.VectorSubcoreMesh(core_axis_name="c", subcore_axis_name="s")
_cparams = pltpu.CompilerParams(needs_layout_passes=False,
                                use_tc_tiling_on_sc=False)


def _mix_body(x_hbm, batch_hbm, w_hbm, out_hbm,
              chunk_v, eloc_v, shared_hist, shared_cmax,
              stripe_v, cmax_v, cm_v,
              offw_v, w_v, buf3_v, buft_v, obuf_v, sem0, sem1, semt):
    cid = lax.axis_index("c")
    sid = lax.axis_index("s")
    wid = sid * NC + cid
    base = wid * SEG_PER
    i16 = lax.iota(jnp.int32, L)

    def zstep(k, _):
        eloc_v[pl.ds(k * L, L)] = jnp.zeros((L,), jnp.int32)
        return 0
    lax.fori_loop(0, S // L, zstep, 0)

    last = sid == NS - 1

    @pl.when(jnp.logical_not(last))
    def _():
        pltpu.sync_copy(batch_hbm.at[pl.ds(sid * CA, CA + L)],
                        chunk_v.at[pl.ds(0, CA + L)])

    @pl.when(last)
    def _():
        pltpu.sync_copy(batch_hbm.at[pl.ds((NS - 1) * CA, CA_LAST)],
                        chunk_v.at[pl.ds(0, CA_LAST)])

    gbase = sid * CA
    nsteps = jnp.where(last, STEPS_LAST, STEPS)
    nquads = nsteps // 4

    def stepn(n):
        def body(q, _):
            i = n * q
            val = gbase + i * L + 1 + i16
            for u in range(n):
                a = chunk_v[pl.ds((i + u) * L, L)]
                b = chunk_v[pl.ds((i + u) * L + 1, L)]
                plsc.store_scatter(eloc_v, [a], val + u * L, mask=a != b)
            return 0
        return body
    lax.fori_loop(0, nquads, stepn(4), 0)
    lax.fori_loop(4 * nquads, nsteps, stepn(1), 0)

    @pl.when(last)
    def _():
        vlast = chunk_v[pl.ds(CA_LAST - L, L)]
        blast = jnp.max(jnp.where(i16 == L - 1, vlast, 0))
        plsc.store_scatter(eloc_v, [blast + jnp.zeros((L,), jnp.int32)],
                           jnp.full((L,), N, jnp.int32), mask=i16 == 0)

    pltpu.sync_copy(eloc_v, shared_hist.at[sid])
    plsc.subcore_barrier()

    pltpu.sync_copy(shared_hist.at[:, pl.ds(sid * 64, 64)], stripe_v)
    for k in range(4):
        m = stripe_v[0, pl.ds(k * L, L)]
        for rr in range(1, NS):
            m = jnp.maximum(m, stripe_v[rr, pl.ds(k * L, L)])
        cmax_v[pl.ds(k * L, L)] = m
    pltpu.sync_copy(cmax_v, shared_cmax.at[pl.ds(sid * 64, 64)])
    plsc.subcore_barrier()

    pltpu.sync_copy(shared_cmax, cm_v.at[pl.ds(0, S)])

    def pmax(i, m):
        return jnp.maximum(m, cm_v[pl.ds(i * L, L)])
    mvec = lax.fori_loop(0, base // L, pmax, jnp.zeros((L,), jnp.int32))
    carry = jnp.max(mvec)
    plsc.store_scatter(offw_v, [i16], jnp.full((L,), 0, jnp.int32) + carry,
                       mask=i16 == 0)
    for k in range(3):
        c = jnp.maximum(plsc.cummax(cm_v[pl.ds(base + k * L, L)]), carry)
        idx = k * L + 1 + i16
        plsc.store_scatter(offw_v, [idx], c, mask=idx < 34)
        carry = jnp.max(c)

    pltpu.sync_copy(w_hbm, w_v)
    wv = plsc.load_gather(w_v, [jnp.minimum(i16, 2)])
    w0 = jnp.sum(jnp.where(i16 == 0, wv, 0.0))
    w1 = jnp.sum(jnp.where(i16 == 1, wv, 0.0))
    w2 = jnp.sum(jnp.where(i16 == 2, wv, 0.0))

    def get_off(j):
        chunk = offw_v[pl.ds((j // L) * L, L)]
        return jnp.max(jnp.where(i16 == j % L, chunk, 0))

    acc0 = (tuple(jnp.zeros((L,), jnp.float32) for _ in range(R)),
            tuple(jnp.full((L,), -3.0e38, jnp.float32) for _ in range(R)))

    def row_loop(ref, a, b, bufbase, accs):
        def rbody1(row, ac):
            s_t, m_t = ac
            bi = row - bufbase
            s_l, m_l = [], []
            for r in range(R):
                v = ref[bi, pl.ds(r * L, L)]
                s_l.append(s_t[r] + v)
                m_l.append(jnp.maximum(m_t[r], v))
            return (tuple(s_l), tuple(m_l))

        def rbody4(k, ac):
            s_t, m_t = ac
            bi = (a - bufbase) + 4 * k
            s_l, m_l = [], []
            for r in range(R):
                v0 = ref[bi, pl.ds(r * L, L)]
                v1 = ref[bi + 1, pl.ds(r * L, L)]
                v2 = ref[bi + 2, pl.ds(r * L, L)]
                v3 = ref[bi + 3, pl.ds(r * L, L)]
                s_l.append(s_t[r] + ((v0 + v1) + (v2 + v3)))
                m_l.append(jnp.maximum(
                    m_t[r], jnp.maximum(jnp.maximum(v0, v1),
                                        jnp.maximum(v2, v3))))
            return (tuple(s_l), tuple(m_l))

        n4 = (b - a) // 4
        accs = lax.fori_loop(0, n4, rbody4, accs)
        return lax.fori_loop(a + 4 * n4, b, rbody1, accs)

    def consume(ref, chi, bufbase, st):
        def w_cond(t):
            s, _, send, _, _ = t
            return jnp.logical_and(s < SEG_PER, send <= chi)

        def w_body(t):
            s, sst, send, cur, accs = t
            accs = row_loop(ref, cur, send, bufbase, accs)
            s_t, m_t = accs
            cnt = send - sst
            cntf = cnt.astype(jnp.float32)
            posf = jnp.where(cnt > 0, jnp.float32(1.0), jnp.float32(0.0))
            denom = jnp.maximum(cntf, 1.0)
            for r in range(R):
                obuf_v[s, pl.ds(r * L, L)] = (w0 * (s_t[r] / denom)
                                              + w1 * (m_t[r] * posf)
                                              + w2 * s_t[r])
            s2 = s + 1
            return (s2, send, get_off(s2 + 1), send, acc0)

        s, sst, send, cur, accs = lax.while_loop(w_cond, w_body, st)
        accs = row_loop(ref, cur, chi, bufbase, accs)
        return (s, sst, send, chi, accs)

    rs = get_off(jnp.int32(0))
    re = get_off(jnp.int32(SEG_PER))
    nf = (re - rs) // K
    ot = jnp.maximum(re - K, 0)

    @pl.when(nf > 0)
    def _():
        pltpu.async_copy(x_hbm.at[pl.ds(rs, K)], buf3_v.at[pl.ds(0, K)], sem0)

    @pl.when(nf > 1)
    def _():
        pltpu.async_copy(x_hbm.at[pl.ds(rs + K, K)],
                         buf3_v.at[pl.ds(K, K)], sem1)

    pltpu.async_copy(x_hbm.at[pl.ds(ot, K)], buft_v, semt)

    st0 = (jnp.int32(0), rs, get_off(jnp.int32(1)), rs, acc0)

    def chunk_body(c, st):
        par = lax.rem(c, 2)

        @pl.when(par == 0)
        def _():
            pltpu.make_async_copy(x_hbm.at[pl.ds(0, K)],
                                  buf3_v.at[pl.ds(0, K)], sem0).wait()

        @pl.when(par == 1)
        def _():
            pltpu.make_async_copy(x_hbm.at[pl.ds(0, K)],
                                  buf3_v.at[pl.ds(K, K)], sem1).wait()

        clo = rs + c * K
        st = consume(buf3_v, clo + K, clo - par * K, st)

        nxt = clo + 2 * K
        issue = nxt < rs + nf * K

        @pl.when(jnp.logical_and(issue, par == 0))
        def _():
            pltpu.async_copy(x_hbm.at[pl.ds(nxt, K)],
                             buf3_v.at[pl.ds(0, K)], sem0)

        @pl.when(jnp.logical_and(issue, par == 1))
        def _():
            pltpu.async_copy(x_hbm.at[pl.ds(nxt, K)],
                             buf3_v.at[pl.ds(K, K)], sem1)

        return st

    st = lax.fori_loop(0, nf, chunk_body, st0)

    pltpu.make_async_copy(x_hbm.at[pl.ds(0, K)], buft_v, semt).wait()
    consume(buft_v, re, ot, st)

    pltpu.sync_copy(obuf_v, out_hbm.at[pl.ds(base, SEG_PER)])


_mix_call = pl.kernel(
    _mix_body,
    out_type=jax.ShapeDtypeStruct((S, H), jnp.float32),
    mesh=_mesh,
    compiler_params=_cparams,
    scratch_types=[
        pltpu.VMEM((CA + L,), jnp.int32),
        pltpu.VMEM((S,), jnp.int32),
        pltpu.VMEM_SHARED((NS, S), jnp.int32),
        pltpu.VMEM_SHARED((S,), jnp.int32),
        pltpu.VMEM((NS, 64), jnp.int32),
        pltpu.VMEM((64,), jnp.int32),
        pltpu.VMEM((S + L,), jnp.int32),
        pltpu.VMEM((48,), jnp.int32),
        pltpu.VMEM((3,), jnp.float32),
        pltpu.VMEM((2 * K, H), jnp.float32),
        pltpu.VMEM((K, H), jnp.float32),
        pltpu.VMEM((SEG_PER, H), jnp.float32),
        pltpu.SemaphoreType.DMA,
        pltpu.SemaphoreType.DMA,
        pltpu.SemaphoreType.DMA,
    ],
)


def kernel(x, batch, mask, weights):
    del mask
    return _mix_call(x, batch, weights)


# speedup vs baseline: 1.0344x; 1.0014x over previous
import jax
import jax.numpy as jnp
from jax import lax
from jax.experimental import pallas as pl
from jax.experimental.pallas import tpu as pltpu
from jax.experimental.pallas import tpu_sc as plsc

N = 100000
H = 128
S = 1024
L = 16
NC = 2
NS = 16
NW = NC * NS
SEG_PER = S // NW
R = H // L

CA = 6272
CA_LAST = N - (NS - 1) * CA
STEPS = CA // L
STEPS_LAST = CA_LAST // L

K = 128

_mesh = plsc.VectorSubcoreMesh(core_axis_name="c", subcore_axis_name="s")
_cparams = pltpu.CompilerParams(needs_layout_passes=False,
                                use_tc_tiling_on_sc=False)


def _mix_body(x_hbm, batch_hbm, w_hbm, out_hbm,
              chunk_v, eloc_v, shared_hist, shared_cmax,
              stripe_v, cmax_v, cm_v,
              offw_v, w_v, buf3_v, buft_v, obuf_v, sem0, sem1, semt):
    cid = lax.axis_index("c")
    sid = lax.axis_index("s")
    wid = sid * NC + cid
    base = wid * SEG_PER
    i16 = lax.iota(jnp.int32, L)

    def zstep(k, _):
        eloc_v[pl.ds(k * L, L)] = jnp.zeros((L,), jnp.int32)
        return 0
    lax.fori_loop(0, S // L, zstep, 0)

    last = sid == NS - 1

    @pl.when(jnp.logical_not(last))
    def _():
        pltpu.sync_copy(batch_hbm.at[pl.ds(sid * CA, CA + L)],
                        chunk_v.at[pl.ds(0, CA + L)])

    @pl.when(last)
    def _():
        pltpu.sync_copy(batch_hbm.at[pl.ds((NS - 1) * CA, CA_LAST)],
                        chunk_v.at[pl.ds(0, CA_LAST)])

    gbase = sid * CA
    nsteps = jnp.where(last, STEPS_LAST, STEPS)
    nquads = nsteps // 4

    def stepn(n):
        def body(q, _):
            i = n * q
            val = gbase + i * L + 1 + i16
            for u in range(n):
                a = chunk_v[pl.ds((i + u) * L, L)]
                b = chunk_v[pl.ds((i + u) * L + 1, L)]
                plsc.store_scatter(eloc_v, [a], val + u * L, mask=a != b)
            return 0
        return body
    lax.fori_loop(0, nquads, stepn(4), 0)
    lax.fori_loop(4 * nquads, nsteps, stepn(1), 0)

    @pl.when(last)
    def _():
        vlast = chunk_v[pl.ds(CA_LAST - L, L)]
        blast = jnp.max(jnp.where(i16 == L - 1, vlast, 0))
        plsc.store_scatter(eloc_v, [blast + jnp.zeros((L,), jnp.int32)],
                           jnp.full((L,), N, jnp.int32), mask=i16 == 0)

    pltpu.sync_copy(eloc_v, shared_hist.at[sid])
    plsc.subcore_barrier()

    pltpu.sync_copy(shared_hist.at[:, pl.ds(sid * 64, 64)], stripe_v)
    for k in range(4):
        m = stripe_v[0, pl.ds(k * L, L)]
        for rr in range(1, NS):
            m = jnp.maximum(m, stripe_v[rr, pl.ds(k * L, L)])
        cmax_v[pl.ds(k * L, L)] = m
    pltpu.sync_copy(cmax_v, shared_cmax.at[pl.ds(sid * 64, 64)])
    plsc.subcore_barrier()

    pltpu.sync_copy(shared_cmax, cm_v.at[pl.ds(0, S)])

    def pmax(i, m):
        return jnp.maximum(m, cm_v[pl.ds(i * L, L)])
    mvec = lax.fori_loop(0, base // L, pmax, jnp.zeros((L,), jnp.int32))
    carry = jnp.max(mvec)
    plsc.store_scatter(offw_v, [i16], jnp.full((L,), 0, jnp.int32) + carry,
                       mask=i16 == 0)
    for k in range(3):
        c = jnp.maximum(plsc.cummax(cm_v[pl.ds(base + k * L, L)]), carry)
        idx = k * L + 1 + i16
        plsc.store_scatter(offw_v, [idx], c, mask=idx < 34)
        carry = jnp.max(c)

    pltpu.sync_copy(w_hbm, w_v)
    wv = plsc.load_gather(w_v, [jnp.minimum(i16, 2)])
    w0 = jnp.sum(jnp.where(i16 == 0, wv, 0.0))
    w1 = jnp.sum(jnp.where(i16 == 1, wv, 0.0))
    w2 = jnp.sum(jnp.where(i16 == 2, wv, 0.0))

    def get_off(j):
        chunk = offw_v[pl.ds((j // L) * L, L)]
        return jnp.max(jnp.where(i16 == j % L, chunk, 0))

    acc0 = (tuple(jnp.zeros((L,), jnp.float32) for _ in range(R)),
            tuple(jnp.full((L,), -3.0e38, jnp.float32) for _ in range(R)))

    def row_loop(ref, a, b, bufbase, accs):
        def rbody1(row, ac):
            s_t, m_t = ac
            bi = row - bufbase
            s_l, m_l = [], []
            for r in range(R):
                v = ref[bi, pl.ds(r * L, L)]
                s_l.append(s_t[r] + v)
                m_l.append(jnp.maximum(m_t[r], v))
            return (tuple(s_l), tuple(m_l))

        def rbody4(k, ac):
            s_t, m_t = ac
            bi = (a - bufbase) + 4 * k
            s_l, m_l = [], []
            for r in range(R):
                v0 = ref[bi, pl.ds(r * L, L)]
                v1 = ref[bi + 1, pl.ds(r * L, L)]
                v2 = ref[bi + 2, pl.ds(r * L, L)]
                v3 = ref[bi + 3, pl.ds(r * L, L)]
                s_l.append(s_t[r] + ((v0 + v1) + (v2 + v3)))
                m_l.append(jnp.maximum(
                    m_t[r], jnp.maximum(jnp.maximum(v0, v1),
                                        jnp.maximum(v2, v3))))
            return (tuple(s_l), tuple(m_l))

        n4 = (b - a) // 4
        accs = lax.fori_loop(0, n4, rbody4, accs)
        return lax.fori_loop(a + 4 * n4, b, rbody1, accs)

    def consume(ref, chi, bufbase, st):
        def w_cond(t):
            s, _, send, _, _ = t
            return jnp.logical_and(s < SEG_PER, send <= chi)

        def w_body(t):
            s, sst, send, cur, accs = t
            accs = row_loop(ref, cur, send, bufbase, accs)
            s_t, m_t = accs
            cnt = send - sst
            cntf = cnt.astype(jnp.float32)
            posf = jnp.where(cnt > 0, jnp.float32(1.0), jnp.float32(0.0))
            denom = jnp.maximum(cntf, 1.0)
            for r in range(R):
                obuf_v[s, pl.ds(r * L, L)] = (w0 * (s_t[r] / denom)
                                              + w1 * (m_t[r] * posf)
                                              + w2 * s_t[r])
            s2 = s + 1
            return (s2, send, get_off(s2 + 1), send, acc0)

        s, sst, send, cur, accs = lax.while_loop(w_cond, w_body, st)
        accs = row_loop(ref, cur, chi, bufbase, accs)
        return (s, sst, send, chi, accs)

    rs = get_off(jnp.int32(0))
    re = get_off(jnp.int32(SEG_PER))
    nf = (re - rs) // K
    ot = jnp.maximum(re - K, 0)

    @pl.when(nf > 0)
    def _():
        pltpu.async_copy(x_hbm.at[pl.ds(rs, K)], buf3_v.at[pl.ds(0, K)], sem0)

    @pl.when(nf > 1)
    def _():
        pltpu.async_copy(x_hbm.at[pl.ds(rs + K, K)],
                         buf3_v.at[pl.ds(K, K)], sem1)

    pltpu.async_copy(x_hbm.at[pl.ds(ot, K)], buft_v, semt)

    st0 = (jnp.int32(0), rs, get_off(jnp.int32(1)), rs, acc0)

    def chunk_body(c, st):
        par = lax.rem(c, 2)

        @pl.when(par == 0)
        def _():
            pltpu.make_async_copy(x_hbm.at[pl.ds(0, K)],
                                  buf3_v.at[pl.ds(0, K)], sem0).wait()

        @pl.when(par == 1)
        def _():
            pltpu.make_async_copy(x_hbm.at[pl.ds(0, K)],
                                  buf3_v.at[pl.ds(K, K)], sem1).wait()

        clo = rs + c * K
        st = consume(buf3_v, clo + K, clo - par * K, st)

        nxt = clo + 2 * K
        issue = nxt < rs + nf * K

        @pl.when(jnp.logical_and(issue, par == 0))
        def _():
            pltpu.async_copy(x_hbm.at[pl.ds(nxt, K)],
                             buf3_v.at[pl.ds(0, K)], sem0)

        @pl.when(jnp.logical_and(issue, par == 1))
        def _():
            pltpu.async_copy(x_hbm.at[pl.ds(nxt, K)],
                             buf3_v.at[pl.ds(K, K)], sem1)

        return st

    st = lax.fori_loop(0, nf, chunk_body, st0)

    pltpu.make_async_copy(x_hbm.at[pl.ds(0, K)], buft_v, semt).wait()
    consume(buft_v, re, ot, st)

    pltpu.sync_copy(obuf_v, out_hbm.at[pl.ds(base, SEG_PER)])


_mix_call = pl.kernel(
    _mix_body,
    out_type=jax.ShapeDtypeStruct((S, H), jnp.float32),
    mesh=_mesh,
    compiler_params=_cparams,
    scratch_types=[
        pltpu.VMEM((CA + L,), jnp.int32),
        pltpu.VMEM((S,), jnp.int32),
        pltpu.VMEM_SHARED((NS, S), jnp.int32),
        pltpu.VMEM_SHARED((S,), jnp.int32),
        pltpu.VMEM((NS, 64), jnp.int32),
        pltpu.VMEM((64,), jnp.int32),
        pltpu.VMEM((S + L,), jnp.int32),
        pltpu.VMEM((48,), jnp.int32),
        pltpu.VMEM((3,), jnp.float32),
        pltpu.VMEM((2 * K, H), jnp.float32),
        pltpu.VMEM((K, H), jnp.float32),
        pltpu.VMEM((SEG_PER, H), jnp.float32),
        pltpu.SemaphoreType.DMA,
        pltpu.SemaphoreType.DMA,
        pltpu.SemaphoreType.DMA,
    ],
)


def kernel(x, batch, mask, weights):
    del mask
    return _mix_call(x, batch, weights)
